# baseline (speedup 1.0000x reference)
import dataclasses
import functools

import jax
import jax.numpy as jnp
from jax import lax
from jax.experimental import pallas as pl
from jax.experimental.pallas import tpu as pltpu
from jax.experimental.pallas import tpu_sc as plsc

B = 1024
N = 100000
L = 16
NC, NS = 2, 16
NW = NC * NS
BPW = B // NW

RB = 8
K = 4
S = B // (RB * K)


def _sc_gather(indexes, labels128, x128):
    mesh = plsc.VectorSubcoreMesh(core_axis_name="c", subcore_axis_name="s")
    cp = pltpu.CompilerParams()
    if "needs_layout_passes" in pltpu.CompilerParams.__dataclass_fields__:
        cp = dataclasses.replace(cp, needs_layout_passes=False)

    @functools.partial(
        pl.kernel,
        out_type=(jax.ShapeDtypeStruct((B,), jnp.int32),
                  jax.ShapeDtypeStruct((B,), jnp.float32)),
        mesh=mesh,
        compiler_params=cp,
        scratch_types=[
            pltpu.VMEM((BPW,), jnp.int32),
            pltpu.VMEM((BPW,), jnp.int32),
            pltpu.VMEM((BPW,), jnp.int32),
            pltpu.VMEM((BPW, 128), jnp.int32),
            pltpu.VMEM((BPW, 128), jnp.float32),
            pltpu.VMEM((BPW,), jnp.int32),
            pltpu.VMEM((BPW,), jnp.float32),
            pltpu.SemaphoreType.DMA,
            pltpu.SemaphoreType.DMA,
        ],
    )
    def k(idx_hbm, lab_hbm, x_hbm, y_hbm, diag_hbm,
          idx_v, lrow_v, xrow_v, lrows_v, xrows_v, y_v, diag_v, sem1, sem2):
        wid = lax.axis_index("s") * NC + lax.axis_index("c")
        base = wid * BPW
        pltpu.sync_copy(idx_hbm.at[pl.ds(base, BPW)], idx_v)
        for j in range(BPW // L):
            idxr = idx_v[pl.ds(j * L, L)]
            b = jax.lax.iota(jnp.int32, L) + (base + j * L)
            flat = b * N + idxr
            lrow_v[pl.ds(j * L, L)] = jax.lax.shift_right_logical(idxr, 7)
            xrow_v[pl.ds(j * L, L)] = jax.lax.shift_right_logical(flat, 7)
        c1 = pltpu.async_copy(lab_hbm.at[lrow_v], lrows_v, sem1)
        c2 = pltpu.async_copy(x_hbm.at[xrow_v], xrows_v, sem2)
        c1.wait()
        c2.wait()
        for j in range(BPW // L):
            idxr = idx_v[pl.ds(j * L, L)]
            b = jax.lax.iota(jnp.int32, L) + (base + j * L)
            flat = b * N + idxr
            rowsel = jax.lax.iota(jnp.int32, L) + j * L
            lane_l = jax.lax.bitwise_and(idxr, 127)
            lane_x = jax.lax.bitwise_and(flat, 127)
            y_v[pl.ds(j * L, L)] = plsc.load_gather(lrows_v, [rowsel, lane_l])
            diag_v[pl.ds(j * L, L)] = plsc.load_gather(xrows_v, [rowsel, lane_x])
        pltpu.sync_copy(y_v, y_hbm.at[pl.ds(base, BPW)])
        pltpu.sync_copy(diag_v, diag_hbm.at[pl.ds(base, BPW)])

    return k(indexes, labels128, x128)


def _sweep_body(*refs):
    d_refs = refs[0:K]
    y_refs = refs[K:2 * K]
    lab_ref = refs[2 * K]
    x_refs = refs[2 * K + 1:3 * K + 1]
    out_ref = refs[3 * K + 1]
    loss_acc = refs[3 * K + 2]
    i = pl.program_id(0)

    @pl.when(i == 0)
    def _init():
        loss_acc[...] = jnp.zeros_like(loss_acc)

    lab = lab_ref[...]
    part = None
    for j in range(K):
        e = jnp.exp(x_refs[j][...])
        pe = jnp.where(lab == y_refs[j][...], e, 0.0)
        ed = jnp.exp(d_refs[j][...])
        p = jnp.sum(pe, axis=1, keepdims=True) - ed
        z = jnp.sum(e, axis=1, keepdims=True) - ed
        prob = p / z
        ok = prob != 0.0
        ll = jnp.where(ok, jnp.log(jnp.where(ok, prob, 1.0)), 0.0)
        s = jnp.sum(ll, axis=0, keepdims=True)
        part = s if part is None else part + s
    loss_acc[...] += part

    @pl.when(i == S - 1)
    def _fin():
        out_ref[...] = -loss_acc[...] / B


def _tc_loss(x, diag, labels, y):
    def rowmap(j):
        return lambda i: (j * S + i, 0)

    grid_spec = pltpu.PrefetchScalarGridSpec(
        num_scalar_prefetch=0,
        grid=(S,),
        in_specs=(
            [pl.BlockSpec((RB, 1), rowmap(j)) for j in range(K)]
            + [pl.BlockSpec((RB, 1), rowmap(j)) for j in range(K)]
            + [pl.BlockSpec((1, N), lambda i: (0, 0))]
            + [pl.BlockSpec((RB, N), rowmap(j)) for j in range(K)]
        ),
        out_specs=pl.BlockSpec((1, 1), lambda i: (0, 0)),
        scratch_shapes=[
            pltpu.VMEM((1, 1), jnp.float32),
        ],
    )
    d2 = diag.reshape(B, 1)
    y2 = y.reshape(B, 1)
    out = pl.pallas_call(
        _sweep_body,
        grid_spec=grid_spec,
        out_shape=jax.ShapeDtypeStruct((1, 1), jnp.float32),
    )(*([d2] * K + [y2] * K + [labels.reshape(1, N)] + [x] * K))
    return out[0, 0]


def kernel(x, indexes, labels):
    npad = -N % 128
    labels128 = jnp.pad(labels, (0, npad)).reshape(-1, 128)
    y, diag = _sc_gather(indexes, labels128, x.reshape(B * N // 128, 128))
    return _tc_loss(x, diag, labels, y)


# speedup vs baseline: 1.5567x; 1.5567x over previous
import dataclasses
import functools

import jax
import jax.numpy as jnp
from jax import lax
from jax.experimental import pallas as pl
from jax.experimental.pallas import tpu as pltpu
from jax.experimental.pallas import tpu_sc as plsc

B = 1024
N = 100000
L = 16
NC, NS = 2, 16
NW = NC * NS
BPW = B // NW

GR = 8
SC_ROWS = (NW // 2) * GR
B_TC = B - SC_ROWS

RB = 8
K = 4
S = B_TC // (RB * K)

CW = 49920
CH = 4992
CHUNKS = CW // CH
CI = CH // L
TAIL0 = 2 * CW
TAIL = N - TAIL0


def _sc_compiler_params():
    cp = pltpu.CompilerParams()
    if "needs_layout_passes" in pltpu.CompilerParams.__dataclass_fields__:
        cp = dataclasses.replace(cp, needs_layout_passes=False)
    return cp


def _sc_gather_y(indexes, labels128):
    mesh = plsc.VectorSubcoreMesh(core_axis_name="c", subcore_axis_name="s")

    @functools.partial(
        pl.kernel,
        out_type=jax.ShapeDtypeStruct((B,), jnp.int32),
        mesh=mesh,
        compiler_params=_sc_compiler_params(),
        scratch_types=[
            pltpu.VMEM((BPW,), jnp.int32),
            pltpu.VMEM((BPW,), jnp.int32),
            pltpu.VMEM((BPW, 128), jnp.int32),
            pltpu.VMEM((BPW,), jnp.int32),
            pltpu.SemaphoreType.DMA,
        ],
    )
    def k(idx_hbm, lab_hbm, y_hbm, idx_v, row_v, rows_v, y_v, sem):
        wid = lax.axis_index("s") * NC + lax.axis_index("c")
        base = wid * BPW
        pltpu.sync_copy(idx_hbm.at[pl.ds(base, BPW)], idx_v)
        for j in range(BPW // L):
            idxr = idx_v[pl.ds(j * L, L)]
            row_v[pl.ds(j * L, L)] = jax.lax.shift_right_logical(idxr, 7)
        pltpu.async_copy(lab_hbm.at[row_v], rows_v, sem).wait()
        for j in range(BPW // L):
            idxr = idx_v[pl.ds(j * L, L)]
            lane = jax.lax.bitwise_and(idxr, 127)
            rowsel = jax.lax.iota(jnp.int32, L) + j * L
            y_v[pl.ds(j * L, L)] = plsc.load_gather(rows_v, [rowsel, lane])
        pltpu.sync_copy(y_v, y_hbm.at[pl.ds(base, BPW)])

    return k(indexes, labels128)


def _sc_sweep(x, indexes, labels, y):
    mesh = plsc.VectorSubcoreMesh(core_axis_name="c", subcore_axis_name="s")

    @functools.partial(
        pl.kernel,
        out_type=(jax.ShapeDtypeStruct((2 * SC_ROWS, L), jnp.float32),
                  jax.ShapeDtypeStruct((2 * SC_ROWS, L), jnp.float32)),
        mesh=mesh,
        compiler_params=_sc_compiler_params(),
        scratch_types=(
            [pltpu.VMEM((GR,), jnp.int32)] * 2
            + [pltpu.VMEM((GR, CH), jnp.float32)] * 2
            + [pltpu.VMEM((CH,), jnp.int32)] * 2
            + [pltpu.VMEM((GR, L), jnp.float32)] * 2
            + [pltpu.SemaphoreType.DMA] * 2
        ),
    )
    def k(x_hbm, idx_hbm, lab_hbm, y_hbm, p_hbm, z_hbm,
          idx8, y8, xa, xb, la, lb, pacc, zacc, sema, semb):
        wid = lax.axis_index("s") * NC + lax.axis_index("c")
        group = wid // 2
        half = wid % 2
        rows0 = B_TC + group * GR
        c0 = half * CW
        pltpu.sync_copy(idx_hbm.at[pl.ds(rows0, GR)], idx8)
        pltpu.sync_copy(y_hbm.at[pl.ds(rows0, GR)], y8)

        xbufs = (xa, xb)
        lbufs = (la, lb)

        def xcopy(c, par, w):
            return pltpu.make_async_copy(
                x_hbm.at[pl.ds(rows0, GR), pl.ds(c, w)],
                xbufs[par].at[:, pl.ds(0, w)], sema)

        def lcopy(c, par, w):
            return pltpu.make_async_copy(
                lab_hbm.at[pl.ds(c, w)], lbufs[par].at[pl.ds(0, w)], semb)

        for q in range(GR):
            pacc[q, :] = jnp.zeros((L,), jnp.float32)
            zacc[q, :] = jnp.zeros((L,), jnp.float32)

        zero16 = jax.lax.iota(jnp.int32, L) * 0
        idxs = [plsc.load_gather(idx8, [zero16 + q]) for q in range(GR)]
        ys = [plsc.load_gather(y8, [zero16 + q]) for q in range(GR)]
        iota = jax.lax.iota(jnp.int32, L)

        def compute(par, cbase, iters):
            lbuf = lbufs[par]
            xbuf = xbufs[par]

            @pl.loop(0, iters)
            def _(i):
                off = i * L
                lv = lbuf[pl.ds(off, L)]
                col = iota + cbase + off
                for q in range(GR):
                    xv = xbuf[q, pl.ds(off, L)]
                    e = jnp.exp(xv)
                    e = jnp.where(col != idxs[q], e, 0.0)
                    pe = jnp.where(lv == ys[q], e, 0.0)
                    zacc[q, :] = zacc[q, :] + e
                    pacc[q, :] = pacc[q, :] + pe

        xcopy(c0, 0, CH).start()
        lcopy(c0, 0, CH).start()
        for c in range(CHUNKS):
            par = c % 2
            if c + 1 < CHUNKS:
                xcopy(c0 + (c + 1) * CH, 1 - par, CH).start()
                lcopy(c0 + (c + 1) * CH, 1 - par, CH).start()
            xcopy(0, par, CH).wait()
            lcopy(0, par, CH).wait()
            compute(par, c0 + c * CH, CI)

        obase = half * SC_ROWS + group * GR
        pltpu.sync_copy(pacc, p_hbm.at[pl.ds(obase, GR)])
        pltpu.sync_copy(zacc, z_hbm.at[pl.ds(obase, GR)])

    return k(x, indexes, labels, y)


def _tc_sweep_body(*refs):
    idx_refs = refs[0:K]
    y_refs = refs[K:2 * K]
    lab_ref = refs[2 * K]
    x_refs = refs[2 * K + 1:3 * K + 1]
    out_ref = refs[3 * K + 1]
    loss_acc = refs[3 * K + 2]
    i = pl.program_id(0)

    @pl.when(i == 0)
    def _init():
        loss_acc[...] = jnp.zeros_like(loss_acc)

    lab = lab_ref[...]
    col = lax.broadcasted_iota(jnp.int32, (1, N), 1)
    part = None
    for j in range(K):
        xe = jnp.exp(x_refs[j][...])
        keep = col != idx_refs[j][...]
        e = jnp.where(keep, xe, 0.0)
        pe = jnp.where(lab == y_refs[j][...], e, 0.0)
        p = jnp.sum(pe, axis=1, keepdims=True)
        z = jnp.sum(e, axis=1, keepdims=True)
        prob = p / z
        ok = prob != 0.0
        ll = jnp.where(ok, jnp.log(jnp.where(ok, prob, 1.0)), 0.0)
        s = jnp.sum(ll, axis=0, keepdims=True)
        part = s if part is None else part + s
    loss_acc[...] += part

    @pl.when(i == S - 1)
    def _fin():
        out_ref[...] = loss_acc[...]


def _tc_sweep(x, indexes, labels, y):
    def rowmap(j):
        return lambda i: (j * S + i, 0)

    grid_spec = pltpu.PrefetchScalarGridSpec(
        num_scalar_prefetch=0,
        grid=(S,),
        in_specs=(
            [pl.BlockSpec((RB, 1), rowmap(j)) for j in range(K)]
            + [pl.BlockSpec((RB, 1), rowmap(j)) for j in range(K)]
            + [pl.BlockSpec((1, N), lambda i: (0, 0))]
            + [pl.BlockSpec((RB, N), rowmap(j)) for j in range(K)]
        ),
        out_specs=pl.BlockSpec((1, 1), lambda i: (0, 0)),
        scratch_shapes=[
            pltpu.VMEM((1, 1), jnp.float32),
        ],
    )
    idx2 = indexes.reshape(B, 1)
    y2 = y.reshape(B, 1)
    return pl.pallas_call(
        _tc_sweep_body,
        grid_spec=grid_spec,
        out_shape=jax.ShapeDtypeStruct((1, 1), jnp.float32),
    )(*([idx2] * K + [y2] * K + [labels.reshape(1, N)] + [x] * K))


def _combine_body(tc_ref, p_ref, z_ref, corner_ref, ltail_ref, ysc_ref,
                  isc_ref, out_ref):
    e = jnp.exp(corner_ref[...])
    col = lax.broadcasted_iota(jnp.int32, (1, TAIL), 1) + TAIL0
    e = jnp.where(col != isc_ref[...], e, 0.0)
    pe = jnp.where(ltail_ref[...] == ysc_ref[...], e, 0.0)
    ph = p_ref[0:SC_ROWS, :] + p_ref[SC_ROWS:2 * SC_ROWS, :]
    zh = z_ref[0:SC_ROWS, :] + z_ref[SC_ROWS:2 * SC_ROWS, :]
    p = jnp.sum(ph, axis=1, keepdims=True) + jnp.sum(pe, axis=1, keepdims=True)
    z = jnp.sum(zh, axis=1, keepdims=True) + jnp.sum(e, axis=1, keepdims=True)
    prob = p / z
    ok = prob != 0.0
    ll = jnp.where(ok, jnp.log(jnp.where(ok, prob, 1.0)), 0.0)
    tot = tc_ref[...] + jnp.sum(ll, axis=0, keepdims=True)
    out_ref[...] = -tot / B


def _combine(tc_part, sc_p, sc_z, corner, ltail, ysc, isc):
    out = pl.pallas_call(
        _combine_body,
        out_shape=jax.ShapeDtypeStruct((1, 1), jnp.float32),
    )(tc_part, sc_p, sc_z, corner, ltail, ysc, isc)
    return out[0, 0]


def kernel(x, indexes, labels):
    npad = -N % 128
    labels128 = jnp.pad(labels, (0, npad)).reshape(-1, 128)
    y = _sc_gather_y(indexes, labels128)
    tc_part = _tc_sweep(x, indexes, labels, y)
    sc_p, sc_z = _sc_sweep(x, indexes, labels, y)
    corner = lax.slice(x, (B_TC, TAIL0), (B, N))
    ltail = labels[TAIL0:].reshape(1, TAIL)
    ysc = y[B_TC:].reshape(SC_ROWS, 1)
    isc = indexes[B_TC:].reshape(SC_ROWS, 1)
    return _combine(tc_part, sc_p, sc_z, corner, ltail, ysc, isc)


# speedup vs baseline: 2.0247x; 1.3007x over previous
import dataclasses
import functools

import jax
import jax.numpy as jnp
from jax import lax
from jax.experimental import pallas as pl
from jax.experimental.pallas import tpu as pltpu
from jax.experimental.pallas import tpu_sc as plsc

B = 1024
N = 100000
L = 16
NC, NS = 2, 16
NW = NC * NS
BPW = B // NW

GR = 8
WPG = 4
SC_ROWS = (NW // WPG) * GR
B_TC = B - SC_ROWS

RB = 8
K = 4
S = B_TC // (RB * K)

CW = 24960
CH = 4992
CHUNKS = CW // CH
CI = CH // L
TAIL0 = WPG * CW
TAIL = N - TAIL0


def _sc_compiler_params():
    cp = pltpu.CompilerParams()
    if "needs_layout_passes" in pltpu.CompilerParams.__dataclass_fields__:
        cp = dataclasses.replace(cp, needs_layout_passes=False)
    return cp


def _sc_gather_y(indexes, labels128):
    mesh = plsc.VectorSubcoreMesh(core_axis_name="c", subcore_axis_name="s")

    @functools.partial(
        pl.kernel,
        out_type=jax.ShapeDtypeStruct((B,), jnp.int32),
        mesh=mesh,
        compiler_params=_sc_compiler_params(),
        scratch_types=[
            pltpu.VMEM((BPW,), jnp.int32),
            pltpu.VMEM((BPW,), jnp.int32),
            pltpu.VMEM((BPW, 128), jnp.int32),
            pltpu.VMEM((BPW,), jnp.int32),
            pltpu.SemaphoreType.DMA,
        ],
    )
    def k(idx_hbm, lab_hbm, y_hbm, idx_v, row_v, rows_v, y_v, sem):
        wid = lax.axis_index("s") * NC + lax.axis_index("c")
        base = wid * BPW
        pltpu.sync_copy(idx_hbm.at[pl.ds(base, BPW)], idx_v)
        for j in range(BPW // L):
            idxr = idx_v[pl.ds(j * L, L)]
            row_v[pl.ds(j * L, L)] = jax.lax.shift_right_logical(idxr, 7)
        pltpu.async_copy(lab_hbm.at[row_v], rows_v, sem).wait()
        for j in range(BPW // L):
            idxr = idx_v[pl.ds(j * L, L)]
            lane = jax.lax.bitwise_and(idxr, 127)
            rowsel = jax.lax.iota(jnp.int32, L) + j * L
            y_v[pl.ds(j * L, L)] = plsc.load_gather(rows_v, [rowsel, lane])
        pltpu.sync_copy(y_v, y_hbm.at[pl.ds(base, BPW)])

    return k(indexes, labels128)


def _sc_sweep(x, indexes, labels, y):
    mesh = plsc.VectorSubcoreMesh(core_axis_name="c", subcore_axis_name="s")

    @functools.partial(
        pl.kernel,
        out_type=(jax.ShapeDtypeStruct((WPG * SC_ROWS, L), jnp.float32),
                  jax.ShapeDtypeStruct((WPG * SC_ROWS, L), jnp.float32)),
        mesh=mesh,
        compiler_params=_sc_compiler_params(),
        scratch_types=(
            [pltpu.VMEM((GR,), jnp.int32)] * 2
            + [pltpu.VMEM((GR, CH), jnp.float32)] * 2
            + [pltpu.VMEM((CH,), jnp.int32)] * 2
            + [pltpu.VMEM((GR, L), jnp.float32)] * 2
            + [pltpu.SemaphoreType.DMA] * 2
        ),
    )
    def k(x_hbm, idx_hbm, lab_hbm, y_hbm, p_hbm, z_hbm,
          idx8, y8, xa, xb, la, lb, pacc, zacc, sema, semb):
        wid = lax.axis_index("s") * NC + lax.axis_index("c")
        group = wid // WPG
        strip = wid % WPG
        rows0 = B_TC + group * GR
        c0 = strip * CW
        pltpu.sync_copy(idx_hbm.at[pl.ds(rows0, GR)], idx8)
        pltpu.sync_copy(y_hbm.at[pl.ds(rows0, GR)], y8)

        xbufs = (xa, xb)
        lbufs = (la, lb)

        def xcopy(c, par, w):
            return pltpu.make_async_copy(
                x_hbm.at[pl.ds(rows0, GR), pl.ds(c, w)],
                xbufs[par].at[:, pl.ds(0, w)], sema)

        def lcopy(c, par, w):
            return pltpu.make_async_copy(
                lab_hbm.at[pl.ds(c, w)], lbufs[par].at[pl.ds(0, w)], semb)

        for q in range(GR):
            pacc[q, :] = jnp.zeros((L,), jnp.float32)
            zacc[q, :] = jnp.zeros((L,), jnp.float32)

        zero16 = jax.lax.iota(jnp.int32, L) * 0
        idxs = [plsc.load_gather(idx8, [zero16 + q]) for q in range(GR)]
        ys = [plsc.load_gather(y8, [zero16 + q]) for q in range(GR)]
        iota = jax.lax.iota(jnp.int32, L)

        def compute(par, cbase, iters):
            lbuf = lbufs[par]
            xbuf = xbufs[par]

            @pl.loop(0, iters)
            def _(i):
                off = i * L
                lv = lbuf[pl.ds(off, L)]
                col = iota + cbase + off
                for q in range(GR):
                    xv = xbuf[q, pl.ds(off, L)]
                    e = jnp.exp(xv)
                    e = jnp.where(col != idxs[q], e, 0.0)
                    pe = jnp.where(lv == ys[q], e, 0.0)
                    zacc[q, :] = zacc[q, :] + e
                    pacc[q, :] = pacc[q, :] + pe

        xcopy(c0, 0, CH).start()
        lcopy(c0, 0, CH).start()
        for c in range(CHUNKS):
            par = c % 2
            if c + 1 < CHUNKS:
                xcopy(c0 + (c + 1) * CH, 1 - par, CH).start()
                lcopy(c0 + (c + 1) * CH, 1 - par, CH).start()
            xcopy(0, par, CH).wait()
            lcopy(0, par, CH).wait()
            compute(par, c0 + c * CH, CI)

        obase = strip * SC_ROWS + group * GR
        pltpu.sync_copy(pacc, p_hbm.at[pl.ds(obase, GR)])
        pltpu.sync_copy(zacc, z_hbm.at[pl.ds(obase, GR)])

    return k(x, indexes, labels, y)


def _tc_sweep_body(*refs):
    idx_refs = refs[0:K]
    y_refs = refs[K:2 * K]
    lab_ref = refs[2 * K]
    x_refs = refs[2 * K + 1:3 * K + 1]
    out_ref = refs[3 * K + 1]
    loss_acc = refs[3 * K + 2]
    i = pl.program_id(0)

    @pl.when(i == 0)
    def _init():
        loss_acc[...] = jnp.zeros_like(loss_acc)

    lab = lab_ref[...]
    col = lax.broadcasted_iota(jnp.int32, (1, N), 1)
    part = None
    for j in range(K):
        xe = jnp.exp(x_refs[j][...])
        keep = col != idx_refs[j][...]
        e = jnp.where(keep, xe, 0.0)
        pe = jnp.where(lab == y_refs[j][...], e, 0.0)
        p = jnp.sum(pe, axis=1, keepdims=True)
        z = jnp.sum(e, axis=1, keepdims=True)
        prob = p / z
        ok = prob != 0.0
        ll = jnp.where(ok, jnp.log(jnp.where(ok, prob, 1.0)), 0.0)
        s = jnp.sum(ll, axis=0, keepdims=True)
        part = s if part is None else part + s
    loss_acc[...] += part

    @pl.when(i == S - 1)
    def _fin():
        out_ref[...] = loss_acc[...]


def _tc_sweep(x, indexes, labels, y):
    def rowmap(j):
        return lambda i: (j * S + i, 0)

    grid_spec = pltpu.PrefetchScalarGridSpec(
        num_scalar_prefetch=0,
        grid=(S,),
        in_specs=(
            [pl.BlockSpec((RB, 1), rowmap(j)) for j in range(K)]
            + [pl.BlockSpec((RB, 1), rowmap(j)) for j in range(K)]
            + [pl.BlockSpec((1, N), lambda i: (0, 0))]
            + [pl.BlockSpec((RB, N), rowmap(j)) for j in range(K)]
        ),
        out_specs=pl.BlockSpec((1, 1), lambda i: (0, 0)),
        scratch_shapes=[
            pltpu.VMEM((1, 1), jnp.float32),
        ],
    )
    idx2 = indexes.reshape(B, 1)
    y2 = y.reshape(B, 1)
    return pl.pallas_call(
        _tc_sweep_body,
        grid_spec=grid_spec,
        out_shape=jax.ShapeDtypeStruct((1, 1), jnp.float32),
    )(*([idx2] * K + [y2] * K + [labels.reshape(1, N)] + [x] * K))


def _combine_body(tc_ref, p_ref, z_ref, corner_ref, ltail_ref, ysc_ref,
                  isc_ref, out_ref):
    e = jnp.exp(corner_ref[...])
    col = lax.broadcasted_iota(jnp.int32, (1, TAIL), 1) + TAIL0
    e = jnp.where(col != isc_ref[...], e, 0.0)
    pe = jnp.where(ltail_ref[...] == ysc_ref[...], e, 0.0)
    ph = p_ref[0:SC_ROWS, :]
    zh = z_ref[0:SC_ROWS, :]
    for h in range(1, WPG):
        ph = ph + p_ref[h * SC_ROWS:(h + 1) * SC_ROWS, :]
        zh = zh + z_ref[h * SC_ROWS:(h + 1) * SC_ROWS, :]
    p = jnp.sum(ph, axis=1, keepdims=True) + jnp.sum(pe, axis=1, keepdims=True)
    z = jnp.sum(zh, axis=1, keepdims=True) + jnp.sum(e, axis=1, keepdims=True)
    prob = p / z
    ok = prob != 0.0
    ll = jnp.where(ok, jnp.log(jnp.where(ok, prob, 1.0)), 0.0)
    tot = tc_ref[...] + jnp.sum(ll, axis=0, keepdims=True)
    out_ref[...] = -tot / B


def _combine(tc_part, sc_p, sc_z, corner, ltail, ysc, isc):
    out = pl.pallas_call(
        _combine_body,
        out_shape=jax.ShapeDtypeStruct((1, 1), jnp.float32),
    )(tc_part, sc_p, sc_z, corner, ltail, ysc, isc)
    return out[0, 0]


def kernel(x, indexes, labels):
    npad = -N % 128
    labels128 = jnp.pad(labels, (0, npad)).reshape(-1, 128)
    y = _sc_gather_y(indexes, labels128)
    tc_part = _tc_sweep(x, indexes, labels, y)
    sc_p, sc_z = _sc_sweep(x, indexes, labels, y)
    corner = lax.slice(x, (B_TC, TAIL0), (B, N))
    ltail = labels[TAIL0:].reshape(1, TAIL)
    ysc = y[B_TC:].reshape(SC_ROWS, 1)
    isc = indexes[B_TC:].reshape(SC_ROWS, 1)
    return _combine(tc_part, sc_p, sc_z, corner, ltail, ysc, isc)


# speedup vs baseline: 2.0932x; 1.0338x over previous
import dataclasses
import functools

import jax
import jax.numpy as jnp
from jax import lax
from jax.experimental import pallas as pl
from jax.experimental.pallas import tpu as pltpu
from jax.experimental.pallas import tpu_sc as plsc

B = 1024
N = 100000
L = 16
NC, NS = 2, 16
NW = NC * NS
BPW = B // NW

TN = 3072
GRID = -(-N // TN)


def _sc_gather_y(indexes, labels128):
    mesh = plsc.VectorSubcoreMesh(core_axis_name="c", subcore_axis_name="s")
    cp = pltpu.CompilerParams()
    if "needs_layout_passes" in pltpu.CompilerParams.__dataclass_fields__:
        cp = dataclasses.replace(cp, needs_layout_passes=False)

    @functools.partial(
        pl.kernel,
        out_type=jax.ShapeDtypeStruct((B,), jnp.int32),
        mesh=mesh,
        compiler_params=cp,
        scratch_types=[
            pltpu.VMEM((BPW,), jnp.int32),
            pltpu.VMEM((BPW,), jnp.int32),
            pltpu.VMEM((BPW, 128), jnp.int32),
            pltpu.VMEM((BPW,), jnp.int32),
            pltpu.SemaphoreType.DMA,
        ],
    )
    def k(idx_hbm, lab_hbm, y_hbm, idx_v, row_v, rows_v, y_v, sem):
        wid = lax.axis_index("s") * NC + lax.axis_index("c")
        base = wid * BPW
        pltpu.sync_copy(idx_hbm.at[pl.ds(base, BPW)], idx_v)
        for j in range(BPW // L):
            idxr = idx_v[pl.ds(j * L, L)]
            row_v[pl.ds(j * L, L)] = jax.lax.shift_right_logical(idxr, 7)
        pltpu.async_copy(lab_hbm.at[row_v], rows_v, sem).wait()
        for j in range(BPW // L):
            idxr = idx_v[pl.ds(j * L, L)]
            lane = jax.lax.bitwise_and(idxr, 127)
            rowsel = jax.lax.iota(jnp.int32, L) + j * L
            y_v[pl.ds(j * L, L)] = plsc.load_gather(rows_v, [rowsel, lane])
        pltpu.sync_copy(y_v, y_hbm.at[pl.ds(base, BPW)])

    return k(indexes, labels128)


def _sweep_body(idx_ref, y_ref, lab_ref, x_ref, out_ref, p_acc, z_acc):
    i = pl.program_id(0)

    @pl.when(i == 0)
    def _init():
        p_acc[...] = jnp.zeros_like(p_acc)
        z_acc[...] = jnp.zeros_like(z_acc)

    xe = jnp.exp(x_ref[...])
    col = lax.broadcasted_iota(jnp.int32, (1, TN), 1) + i * TN
    keep = (col < N) & (col != idx_ref[...])
    e = jnp.where(keep, xe, 0.0)
    pe = jnp.where(lab_ref[...] == y_ref[...], e, 0.0)

    def fold128(t):
        s = t[:, 0:128]
        for k in range(1, TN // 128):
            s = s + t[:, k * 128:(k + 1) * 128]
        return s

    p_acc[...] += fold128(pe)
    z_acc[...] += fold128(e)

    @pl.when(i == GRID - 1)
    def _fin():
        p = jnp.sum(p_acc[...], axis=1, keepdims=True)
        z = jnp.sum(z_acc[...], axis=1, keepdims=True)
        prob = p / z
        ok = prob != 0.0
        ll = jnp.where(ok, jnp.log(jnp.where(ok, prob, 1.0)), 0.0)
        out_ref[...] = -jnp.sum(ll, axis=0, keepdims=True) / B


def _tc_loss(x, indexes, labels, y):
    grid_spec = pltpu.PrefetchScalarGridSpec(
        num_scalar_prefetch=0,
        grid=(GRID,),
        in_specs=[
            pl.BlockSpec((B, 1), lambda i: (0, 0)),
            pl.BlockSpec((B, 1), lambda i: (0, 0)),
            pl.BlockSpec((1, TN), lambda i: (0, i)),
            pl.BlockSpec((B, TN), lambda i: (0, i)),
        ],
        out_specs=pl.BlockSpec((1, 1), lambda i: (0, 0)),
        scratch_shapes=[
            pltpu.VMEM((B, 128), jnp.float32),
            pltpu.VMEM((B, 128), jnp.float32),
        ],
    )
    out = pl.pallas_call(
        _sweep_body,
        grid_spec=grid_spec,
        out_shape=jax.ShapeDtypeStruct((1, 1), jnp.float32),
    )(indexes.reshape(B, 1), y.reshape(B, 1), labels.reshape(1, N), x)
    return out[0, 0]


def kernel(x, indexes, labels):
    npad = -N % 128
    labels128 = jnp.pad(labels, (0, npad)).reshape(-1, 128)
    y = _sc_gather_y(indexes, labels128)
    return _tc_loss(x, indexes, labels, y)


# speedup vs baseline: 2.1172x; 1.0115x over previous
import dataclasses
import functools

import jax
import jax.numpy as jnp
from jax import lax
from jax.experimental import pallas as pl
from jax.experimental.pallas import tpu as pltpu
from jax.experimental.pallas import tpu_sc as plsc

B = 1024
N = 100000
L = 16
NC, NS = 2, 16
NW = NC * NS
BPW = B // NW

TN = 3072
GRID = -(-N // TN)


def _sc_gather_y(indexes, labels128):
    mesh = plsc.VectorSubcoreMesh(core_axis_name="c", subcore_axis_name="s")
    cp = pltpu.CompilerParams()
    if "needs_layout_passes" in pltpu.CompilerParams.__dataclass_fields__:
        cp = dataclasses.replace(cp, needs_layout_passes=False)

    @functools.partial(
        pl.kernel,
        out_type=jax.ShapeDtypeStruct((B,), jnp.int32),
        mesh=mesh,
        compiler_params=cp,
        scratch_types=[
            pltpu.VMEM((BPW,), jnp.int32),
            pltpu.VMEM((BPW,), jnp.int32),
            pltpu.VMEM((BPW, 128), jnp.int32),
            pltpu.VMEM((BPW,), jnp.int32),
            pltpu.SemaphoreType.DMA,
        ],
    )
    def k(idx_hbm, lab_hbm, y_hbm, idx_v, row_v, rows_v, y_v, sem):
        wid = lax.axis_index("s") * NC + lax.axis_index("c")
        base = wid * BPW
        pltpu.sync_copy(idx_hbm.at[pl.ds(base, BPW)], idx_v)
        for j in range(BPW // L):
            idxr = idx_v[pl.ds(j * L, L)]
            row_v[pl.ds(j * L, L)] = jax.lax.shift_right_logical(idxr, 7)
        pltpu.async_copy(lab_hbm.at[row_v], rows_v, sem).wait()
        for j in range(BPW // L):
            idxr = idx_v[pl.ds(j * L, L)]
            lane = jax.lax.bitwise_and(idxr, 127)
            rowsel = jax.lax.iota(jnp.int32, L) + j * L
            y_v[pl.ds(j * L, L)] = plsc.load_gather(rows_v, [rowsel, lane])
        pltpu.sync_copy(y_v, y_hbm.at[pl.ds(base, BPW)])

    return k(indexes, labels128)


def _sweep_body(idx_ref, y_ref, lab_ref, x_ref, out_ref, p_acc, z_acc):
    i = pl.program_id(0)

    @pl.when(i == 0)
    def _init():
        p_acc[...] = jnp.zeros_like(p_acc)
        z_acc[...] = jnp.zeros_like(z_acc)

    def fold128(t):
        s = t[:, 0:128]
        for k in range(1, TN // 128):
            s = s + t[:, k * 128:(k + 1) * 128]
        return s

    def accumulate(mask_pad):
        xe = jnp.exp(x_ref[...])
        col = lax.broadcasted_iota(jnp.int32, (1, TN), 1) + i * TN
        keep = col != idx_ref[...]
        if mask_pad:
            keep = keep & (col < N)
        e = jnp.where(keep, xe, 0.0)
        pe = jnp.where(lab_ref[...] == y_ref[...], e, 0.0)
        p_acc[...] += fold128(pe)
        z_acc[...] += fold128(e)

    @pl.when(i < GRID - 1)
    def _interior():
        accumulate(False)

    @pl.when(i == GRID - 1)
    def _last():
        accumulate(True)

    @pl.when(i == GRID - 1)
    def _fin():
        p = jnp.sum(p_acc[...], axis=1, keepdims=True)
        z = jnp.sum(z_acc[...], axis=1, keepdims=True)
        prob = p / z
        ok = prob != 0.0
        ll = jnp.where(ok, jnp.log(jnp.where(ok, prob, 1.0)), 0.0)
        out_ref[...] = -jnp.sum(ll, axis=0, keepdims=True) / B


def _tc_loss(x, indexes, labels, y):
    grid_spec = pltpu.PrefetchScalarGridSpec(
        num_scalar_prefetch=0,
        grid=(GRID,),
        in_specs=[
            pl.BlockSpec((B, 1), lambda i: (0, 0)),
            pl.BlockSpec((B, 1), lambda i: (0, 0)),
            pl.BlockSpec((1, TN), lambda i: (0, i)),
            pl.BlockSpec((B, TN), lambda i: (0, i)),
        ],
        out_specs=pl.BlockSpec((1, 1), lambda i: (0, 0)),
        scratch_shapes=[
            pltpu.VMEM((B, 128), jnp.float32),
            pltpu.VMEM((B, 128), jnp.float32),
        ],
    )
    out = pl.pallas_call(
        _sweep_body,
        grid_spec=grid_spec,
        out_shape=jax.ShapeDtypeStruct((1, 1), jnp.float32),
    )(indexes.reshape(B, 1), y.reshape(B, 1), labels.reshape(1, N), x)
    return out[0, 0]


def kernel(x, indexes, labels):
    npad = -N % 128
    labels128 = jnp.pad(labels, (0, npad)).reshape(-1, 128)
    y = _sc_gather_y(indexes, labels128)
    return _tc_loss(x, indexes, labels, y)
